# Initial kernel scaffold; baseline (speedup 1.0000x reference)
import jax
import jax.numpy as jnp
from jax.experimental import pallas as pl


def kernel(x):
    raise NotImplementedError("write your pallas kernel here")


# speedup vs baseline: 14.0140x; 14.0140x over previous
import functools

import jax
import jax.numpy as jnp
from jax.experimental import pallas as pl
from jax.experimental.pallas import tpu as pltpu

_K = 256
_ROWS_PER_BLOCK = 8


def _topk_mask_kernel(x_ref, out_ref, u_ref, k):
    x = x_ref[...]
    s = jax.lax.bitcast_convert_type(x, jnp.uint32)
    u = jnp.where(s >= jnp.uint32(0x80000000), ~s, s | jnp.uint32(0x80000000))
    u_ref[...] = u
    rows = x.shape[0]

    def body(i, lo):
        bit = jax.lax.shift_left(jnp.uint32(1), (31 - i).astype(jnp.uint32))
        cand = lo | bit
        cnt = jnp.sum((u_ref[...] >= cand).astype(jnp.int32), axis=1,
                      keepdims=True)
        return jnp.where(cnt >= k, cand, lo)

    lo = jax.lax.fori_loop(0, 32, body, jnp.zeros((rows, 1), jnp.uint32))
    keep = (u >= lo) & (x > 0.0)
    out_ref[...] = jnp.where(keep, x, 0.0)


def kernel(x):
    n_rows, n_cols = x.shape
    r = _ROWS_PER_BLOCK
    grid = (n_rows // r,)
    return pl.pallas_call(
        functools.partial(_topk_mask_kernel, k=_K),
        grid=grid,
        in_specs=[pl.BlockSpec((r, n_cols), lambda i: (i, 0))],
        out_specs=pl.BlockSpec((r, n_cols), lambda i: (i, 0)),
        out_shape=jax.ShapeDtypeStruct(x.shape, x.dtype),
        scratch_shapes=[pltpu.VMEM((r, n_cols), jnp.uint32)],
    )(x)


# speedup vs baseline: 26.7466x; 1.9086x over previous
import functools

import jax
import jax.numpy as jnp
from jax.experimental import pallas as pl
from jax.experimental.pallas import tpu as pltpu

_K = 256
_ROWS_PER_BLOCK = 16


def _key_to_f32(u):
    s = jnp.where(u >= jnp.uint32(0x80000000), u ^ jnp.uint32(0x80000000), ~u)
    return jax.lax.bitcast_convert_type(s, jnp.float32)


def _topk_mask_kernel(x_ref, out_ref, k):
    rows = x_ref.shape[0]

    def body(_, carry):
        lo, hi = carry
        mid = lo + ((hi - lo) >> jnp.uint32(1))
        t = _key_to_f32(mid)
        cnt = jnp.sum((x_ref[...] >= t).astype(jnp.float32), axis=1,
                      keepdims=True)
        ge = cnt >= k
        return jnp.where(ge, mid, lo), jnp.where(ge, hi, mid)

    lo0 = jnp.zeros((rows, 1), jnp.uint32)
    hi0 = jnp.full((rows, 1), 0xFFFFFFFF, jnp.uint32)
    lo, _ = jax.lax.fori_loop(0, 32, body, (lo0, hi0))
    t = _key_to_f32(lo)
    x = x_ref[...]
    out_ref[...] = jnp.where((x >= t) & (x > 0.0), x, 0.0)


def kernel(x):
    n_rows, n_cols = x.shape
    r = _ROWS_PER_BLOCK
    grid = (n_rows // r,)
    return pl.pallas_call(
        functools.partial(_topk_mask_kernel, k=_K),
        grid=grid,
        in_specs=[pl.BlockSpec((r, n_cols), lambda i: (i, 0))],
        out_specs=pl.BlockSpec((r, n_cols), lambda i: (i, 0)),
        out_shape=jax.ShapeDtypeStruct(x.shape, x.dtype),
    )(x)


# speedup vs baseline: 35.7421x; 1.3363x over previous
import functools

import jax
import jax.numpy as jnp
from jax.experimental import pallas as pl
from jax.experimental.pallas import tpu as pltpu

_K = 256
_ROWS_PER_BLOCK = 32


def _key_to_f32(u):
    s = jnp.where(u >= jnp.uint32(0x80000000), u ^ jnp.uint32(0x80000000), ~u)
    return jax.lax.bitcast_convert_type(s, jnp.float32)


def _f32_to_key(x):
    s = jax.lax.bitcast_convert_type(x, jnp.uint32)
    return jnp.where(s >= jnp.uint32(0x80000000), ~s, s | jnp.uint32(0x80000000))


def _topk_mask_kernel(x_ref, out_ref, k):
    rows = x_ref.shape[0]
    cols = x_ref.shape[1]

    g = jnp.max(x_ref[...].reshape(rows, cols // 256, 256), axis=1)
    lo0 = _f32_to_key(jnp.min(g, axis=1, keepdims=True))
    hi0 = _f32_to_key(jnp.max(g, axis=1, keepdims=True)) + jnp.uint32(1)

    def cond(carry):
        lo, hi = carry
        return jnp.any((hi - lo) > jnp.uint32(1))

    def body(carry):
        lo, hi = carry
        mid = lo + ((hi - lo) >> jnp.uint32(1))
        t = _key_to_f32(mid)
        cnt = jnp.sum((x_ref[...] >= t).astype(jnp.float32), axis=1,
                      keepdims=True)
        ge = cnt >= k
        return jnp.where(ge, mid, lo), jnp.where(ge, hi, mid)

    lo, _ = jax.lax.while_loop(cond, body, (lo0, hi0))
    t = _key_to_f32(lo)
    x = x_ref[...]
    out_ref[...] = jnp.where((x >= t) & (x > 0.0), x, 0.0)


def kernel(x):
    n_rows, n_cols = x.shape
    r = _ROWS_PER_BLOCK
    grid = (n_rows // r,)
    return pl.pallas_call(
        functools.partial(_topk_mask_kernel, k=_K),
        grid=grid,
        in_specs=[pl.BlockSpec((r, n_cols), lambda i: (i, 0))],
        out_specs=pl.BlockSpec((r, n_cols), lambda i: (i, 0)),
        out_shape=jax.ShapeDtypeStruct(x.shape, x.dtype),
    )(x)


# speedup vs baseline: 37.1630x; 1.0398x over previous
import functools

import jax
import jax.numpy as jnp
from jax.experimental import pallas as pl
from jax.experimental.pallas import tpu as pltpu

_K = 256
_ROWS_PER_BLOCK = 32


def _key_to_f32(u):
    s = jnp.where(u >= jnp.uint32(0x80000000), u ^ jnp.uint32(0x80000000), ~u)
    return jax.lax.bitcast_convert_type(s, jnp.float32)


def _f32_to_key(x):
    s = jax.lax.bitcast_convert_type(x, jnp.uint32)
    return jnp.where(s >= jnp.uint32(0x80000000), ~s, s | jnp.uint32(0x80000000))


def _topk_mask_kernel(x_ref, out_ref, k):
    rows = x_ref.shape[0]
    cols = x_ref.shape[1]

    n_sl = cols // 256
    accs = [x_ref[:, 256 * i:256 * (i + 1)] for i in range(4)]
    for i in range(4, n_sl):
        accs[i % 4] = jnp.maximum(accs[i % 4], x_ref[:, 256 * i:256 * (i + 1)])
    g = jnp.maximum(jnp.maximum(accs[0], accs[1]),
                    jnp.maximum(accs[2], accs[3]))
    lo0 = _f32_to_key(jnp.min(g, axis=1, keepdims=True))
    hi0 = _f32_to_key(jnp.max(g, axis=1, keepdims=True)) + jnp.uint32(1)

    def cond(carry):
        lo, hi = carry
        return jnp.any((hi - lo) > jnp.uint32(1))

    def body(carry):
        lo, hi = carry
        mid = lo + ((hi - lo) >> jnp.uint32(1))
        t = _key_to_f32(mid)
        cnt = jnp.sum((x_ref[...] >= t).astype(jnp.float32), axis=1,
                      keepdims=True)
        ge = cnt >= k
        return jnp.where(ge, mid, lo), jnp.where(ge, hi, mid)

    lo, _ = jax.lax.while_loop(cond, body, (lo0, hi0))
    t = _key_to_f32(lo)
    x = x_ref[...]
    out_ref[...] = jnp.where((x >= t) & (x > 0.0), x, 0.0)


def kernel(x):
    n_rows, n_cols = x.shape
    r = _ROWS_PER_BLOCK
    grid = (n_rows // r,)
    return pl.pallas_call(
        functools.partial(_topk_mask_kernel, k=_K),
        grid=grid,
        in_specs=[pl.BlockSpec((r, n_cols), lambda i: (i, 0))],
        out_specs=pl.BlockSpec((r, n_cols), lambda i: (i, 0)),
        out_shape=jax.ShapeDtypeStruct(x.shape, x.dtype),
    )(x)
